# baseline (speedup 1.0000x reference)
import functools

import jax
import jax.numpy as jnp
from jax import lax
from jax.experimental import pallas as pl
from jax.experimental.pallas import tpu as pltpu
from jax.experimental.pallas import tpu_sc as plsc

_LANES = 16
_BB = 128
_NN = 8


@functools.lru_cache(maxsize=None)
def _build_scatter(V, T):
    info = plsc.get_sparse_core_info()
    NC, NS = info.num_cores, info.num_subcores
    NW = NC * NS
    TPW = T // NW
    Vp = ((V + NS * 128 - 1) // (NS * 128)) * NS * 128
    CS = Vp // NS
    assert CS % 128 == 0

    mesh = plsc.VectorSubcoreMesh(core_axis_name="c", subcore_axis_name="s")

    @functools.partial(
        pl.kernel,
        out_type=jax.ShapeDtypeStruct((NC, Vp), jnp.float32),
        mesh=mesh,
        compiler_params=pltpu.CompilerParams(needs_layout_passes=False),
        scratch_types=[
            pltpu.VMEM((TPW,), jnp.int32),
            pltpu.VMEM((TPW,), jnp.float32),
            pltpu.VMEM((CS,), jnp.float32),
            pltpu.VMEM_SHARED((Vp,), jnp.float32),
        ],
    )
    def scatter_kernel(idx_hbm, coeff_hbm, out_hbm, idx_w, coeff_w, zer_v, s_sh):
        cid = lax.axis_index("c")
        sid = lax.axis_index("s")
        row = sid * NC + cid
        pltpu.sync_copy(idx_hbm.at[row], idx_w)
        pltpu.sync_copy(coeff_hbm.at[row], coeff_w)

        z = jnp.zeros((_LANES,), jnp.float32)

        def zero(i, carry):
            zer_v[pl.ds(i * _LANES, _LANES)] = z
            return carry

        lax.fori_loop(0, CS // _LANES, zero, None)
        pltpu.sync_copy(zer_v, s_sh.at[pl.ds(sid * CS, CS)])
        plsc.subcore_barrier()
        pltpu.sync_copy(coeff_w, s_sh.at[idx_w], add=True)
        plsc.subcore_barrier()
        pltpu.sync_copy(s_sh.at[pl.ds(sid * CS, CS)],
                        out_hbm.at[cid].at[pl.ds(sid * CS, CS)])

    return scatter_kernel


@functools.lru_cache(maxsize=None)
def _build_matvec(B, V, NC, Vp):
    NJ = 2
    VB = Vp // NJ
    grid = (B // _BB, NJ)
    tail_valid = V - (Vp - 512)
    dims = (((1,), (1,)), ((), ()))

    def dot(a, b):
        return lax.dot_general(
            a, b, dims,
            precision=lax.Precision.HIGHEST,
            preferred_element_type=jnp.float32)

    def body(x_ref, s_ref, rhs_ref, out_ref, acc_ref):
        j = pl.program_id(1)
        sb = s_ref[0:1, :] + s_ref[1:2, :]
        sbm = jnp.broadcast_to(sb, (_NN, VB))

        @pl.when(j == 0)
        def _():
            acc_ref[...] = dot(x_ref[...], sbm)

        @pl.when(j == NJ - 1)
        def _():
            xb = x_ref[...]
            lane = lax.broadcasted_iota(jnp.int32, (_BB, 512), 1)
            xt = jnp.where(lane < tail_valid, xb[:, VB - 512:], 0.0)
            acc = (acc_ref[...]
                   + dot(xb[:, :VB - 512], sbm[:, :VB - 512])
                   + dot(xt, sbm[:, VB - 512:]))
            out_ref[...] = (acc[:, 0] <= rhs_ref[0]).astype(jnp.int32)

    return pl.pallas_call(
        body,
        grid=grid,
        out_shape=jax.ShapeDtypeStruct((B,), jnp.int32),
        in_specs=[
            pl.BlockSpec((_BB, VB), lambda i, j: (i, j)),
            pl.BlockSpec((NC, VB), lambda i, j: (0, j)),
            pl.BlockSpec(memory_space=pltpu.SMEM),
        ],
        out_specs=pl.BlockSpec((_BB,), lambda i, j: (i,)),
        scratch_shapes=[pltpu.VMEM((_BB, _NN), jnp.float32)],
        compiler_params=pltpu.CompilerParams(
            dimension_semantics=("arbitrary", "arbitrary")),
    )


def kernel(x, coeff_tensor, indices_tensor, rhs):
    B, V = x.shape
    T = indices_tensor.shape[0]
    info = plsc.get_sparse_core_info()
    NW = info.num_cores * info.num_subcores
    idx2 = indices_tensor.reshape(NW, T // NW)
    coeff2 = coeff_tensor.reshape(NW, T // NW)
    s = _build_scatter(V, T)(idx2, coeff2)
    rhs_arr = jnp.full((1,), rhs, dtype=jnp.float32)
    out = _build_matvec(B, V, s.shape[0], s.shape[1])(x, s, rhs_arr)
    return out.astype(bool)


# speedup vs baseline: 1.0016x; 1.0016x over previous
import functools

import jax
import jax.numpy as jnp
from jax import lax
from jax.experimental import pallas as pl
from jax.experimental.pallas import tpu as pltpu
from jax.experimental.pallas import tpu_sc as plsc

_LANES = 16
_KB = 2048
_NN = 8


@functools.lru_cache(maxsize=None)
def _build_scatter(V, T):
    info = plsc.get_sparse_core_info()
    NC, NS = info.num_cores, info.num_subcores
    NW = NC * NS
    TPW = T // NW
    Vp = ((V + NS * 128 - 1) // (NS * 128)) * NS * 128
    CS = Vp // NS
    assert CS % 128 == 0

    mesh = plsc.VectorSubcoreMesh(core_axis_name="c", subcore_axis_name="s")

    @functools.partial(
        pl.kernel,
        out_type=jax.ShapeDtypeStruct((NC, Vp), jnp.float32),
        mesh=mesh,
        compiler_params=pltpu.CompilerParams(needs_layout_passes=False),
        scratch_types=[
            pltpu.VMEM((TPW,), jnp.int32),
            pltpu.VMEM((TPW,), jnp.float32),
            pltpu.VMEM((CS,), jnp.float32),
            pltpu.VMEM_SHARED((Vp,), jnp.float32),
        ],
    )
    def scatter_kernel(idx_hbm, coeff_hbm, out_hbm, idx_w, coeff_w, zer_v, s_sh):
        cid = lax.axis_index("c")
        sid = lax.axis_index("s")
        row = sid * NC + cid
        pltpu.sync_copy(idx_hbm.at[row], idx_w)
        pltpu.sync_copy(coeff_hbm.at[row], coeff_w)

        z = jnp.zeros((_LANES,), jnp.float32)

        def zero(i, carry):
            zer_v[pl.ds(i * _LANES, _LANES)] = z
            return carry

        lax.fori_loop(0, CS // _LANES, zero, None)
        pltpu.sync_copy(zer_v, s_sh.at[pl.ds(sid * CS, CS)])
        plsc.subcore_barrier()
        pltpu.sync_copy(coeff_w, s_sh.at[idx_w], add=True)
        plsc.subcore_barrier()
        pltpu.sync_copy(s_sh.at[pl.ds(sid * CS, CS)],
                        out_hbm.at[cid].at[pl.ds(sid * CS, CS)])

    return scatter_kernel


@functools.lru_cache(maxsize=None)
def _build_matvec(B, V, NC, Vp):
    grid = Vp // _KB
    tail_valid = V - (Vp - 512)
    dims = (((1,), (1,)), ((), ()))

    def dot(a, b):
        return lax.dot_general(
            a, b, dims,
            precision=lax.Precision.HIGHEST,
            preferred_element_type=jnp.float32)

    def body(x_ref, s_ref, rhs_ref, out_ref, acc_ref):
        k = pl.program_id(0)
        sb = s_ref[0:1, :] + s_ref[1:2, :]
        sbm = jnp.broadcast_to(sb, (_NN, _KB))

        @pl.when(k == 0)
        def _():
            acc_ref[...] = jnp.zeros_like(acc_ref)

        @pl.when(k < grid - 1)
        def _():
            acc_ref[...] += dot(x_ref[...], sbm)

        @pl.when(k == grid - 1)
        def _():
            xb = x_ref[...]
            lane = lax.broadcasted_iota(jnp.int32, (B, 512), 1)
            xt = jnp.where(lane < tail_valid, xb[:, _KB - 512:], 0.0)
            acc = (acc_ref[...]
                   + dot(xb[:, :_KB - 512], sbm[:, :_KB - 512])
                   + dot(xt, sbm[:, _KB - 512:]))
            out_ref[...] = (acc[:, 0] <= rhs_ref[0]).astype(jnp.int32)

    return pl.pallas_call(
        body,
        grid=(grid,),
        out_shape=jax.ShapeDtypeStruct((B,), jnp.int32),
        in_specs=[
            pl.BlockSpec((B, _KB), lambda k: (0, k)),
            pl.BlockSpec((NC, _KB), lambda k: (0, k)),
            pl.BlockSpec(memory_space=pltpu.SMEM),
        ],
        out_specs=pl.BlockSpec((B,), lambda k: (0,)),
        scratch_shapes=[pltpu.VMEM((B, _NN), jnp.float32)],
        compiler_params=pltpu.CompilerParams(
            dimension_semantics=("arbitrary",)),
    )


def kernel(x, coeff_tensor, indices_tensor, rhs):
    B, V = x.shape
    T = indices_tensor.shape[0]
    info = plsc.get_sparse_core_info()
    NW = info.num_cores * info.num_subcores
    idx2 = indices_tensor.reshape(NW, T // NW)
    coeff2 = coeff_tensor.reshape(NW, T // NW)
    s = _build_scatter(V, T)(idx2, coeff2)
    rhs_arr = jnp.full((1,), rhs, dtype=jnp.float32)
    out = _build_matvec(B, V, s.shape[0], s.shape[1])(x, s, rhs_arr)
    return out.astype(bool)


# speedup vs baseline: 1.1514x; 1.1495x over previous
import functools

import jax
import jax.numpy as jnp
from jax import lax
from jax.experimental import pallas as pl
from jax.experimental.pallas import tpu as pltpu
from jax.experimental.pallas import tpu_sc as plsc

_LANES = 16
_KB = 2048
_NN = 8


@functools.lru_cache(maxsize=None)
def _build_scatter(V, T):
    info = plsc.get_sparse_core_info()
    NC, NS = info.num_cores, info.num_subcores
    NW = NC * NS
    TPW = T // NW
    Vp = ((V + NS * 128 - 1) // (NS * 128)) * NS * 128
    CS = Vp // NS
    assert CS % 128 == 0

    mesh = plsc.VectorSubcoreMesh(core_axis_name="c", subcore_axis_name="s")

    @functools.partial(
        pl.kernel,
        out_type=jax.ShapeDtypeStruct((NC, Vp), jnp.float32),
        mesh=mesh,
        compiler_params=pltpu.CompilerParams(needs_layout_passes=False),
        scratch_types=[
            pltpu.VMEM((TPW,), jnp.int32),
            pltpu.VMEM((TPW,), jnp.float32),
            pltpu.VMEM((CS,), jnp.float32),
            pltpu.VMEM_SHARED((Vp,), jnp.float32),
        ],
    )
    def scatter_kernel(idx_hbm, coeff_hbm, out_hbm, idx_w, coeff_w, zer_v, s_sh):
        cid = lax.axis_index("c")
        sid = lax.axis_index("s")
        row = sid * NC + cid
        pltpu.sync_copy(idx_hbm.at[row], idx_w)
        pltpu.sync_copy(coeff_hbm.at[row], coeff_w)

        z = jnp.zeros((_LANES,), jnp.float32)

        def zero(i, carry):
            zer_v[pl.ds(i * _LANES, _LANES)] = z
            return carry

        lax.fori_loop(0, CS // _LANES, zero, None)
        pltpu.sync_copy(zer_v, s_sh.at[pl.ds(sid * CS, CS)])
        plsc.subcore_barrier()
        pltpu.sync_copy(coeff_w, s_sh.at[idx_w], add=True)
        plsc.subcore_barrier()
        pltpu.sync_copy(s_sh.at[pl.ds(sid * CS, CS)],
                        out_hbm.at[cid].at[pl.ds(sid * CS, CS)])

    return scatter_kernel


@functools.lru_cache(maxsize=None)
def _build_matvec(B, V, NC, T):
    VT = (V + 127) // 128
    tailv = V - (VT - 1) * 128

    def body(bl_ref, fl_ref, x_ref, s_ref, rhs_ref, out_ref, acc_ref):
        k = pl.program_id(0)
        f = fl_ref[k]
        b = bl_ref[k]

        @pl.when(k == 0)
        def _():
            acc_ref[...] = jnp.zeros_like(acc_ref)

        @pl.when((f != 0) & (b != VT - 1))
        def _():
            acc_ref[...] += x_ref[...] * (s_ref[0:1, :] + s_ref[1:2, :])

        @pl.when((f != 0) & (b == VT - 1))
        def _():
            lane = lax.broadcasted_iota(jnp.int32, (B, 128), 1)
            xb = jnp.where(lane < tailv, x_ref[...], 0.0)
            acc_ref[...] += xb * (s_ref[0:1, :] + s_ref[1:2, :])

        @pl.when(k == T - 1)
        def _():
            lhs = jnp.sum(acc_ref[...], axis=1)
            out_ref[...] = (lhs <= rhs_ref[0]).astype(jnp.int32)

    grid_spec = pltpu.PrefetchScalarGridSpec(
        num_scalar_prefetch=2,
        grid=(T,),
        in_specs=[
            pl.BlockSpec((B, 128), lambda k, bl, fl: (0, bl[k])),
            pl.BlockSpec((NC, 128), lambda k, bl, fl: (0, bl[k])),
            pl.BlockSpec(memory_space=pltpu.SMEM),
        ],
        out_specs=pl.BlockSpec((B,), lambda k, bl, fl: (0,)),
        scratch_shapes=[pltpu.VMEM((B, 128), jnp.float32)],
    )
    return pl.pallas_call(
        body,
        grid_spec=grid_spec,
        out_shape=jax.ShapeDtypeStruct((B,), jnp.int32),
        compiler_params=pltpu.CompilerParams(
            dimension_semantics=("arbitrary",)),
    )


def kernel(x, coeff_tensor, indices_tensor, rhs):
    B, V = x.shape
    T = indices_tensor.shape[0]
    info = plsc.get_sparse_core_info()
    NW = info.num_cores * info.num_subcores
    idx2 = indices_tensor.reshape(NW, T // NW)
    coeff2 = coeff_tensor.reshape(NW, T // NW)
    s = _build_scatter(V, T)(idx2, coeff2)
    rhs_arr = jnp.full((1,), rhs, dtype=jnp.float32)
    cbs = jnp.sort(indices_tensor // 128).astype(jnp.int32)
    first = jnp.concatenate(
        [jnp.ones((1,), bool), cbs[1:] != cbs[:-1]])
    rank = jnp.cumsum(first) - 1
    blist = jnp.zeros((T,), jnp.int32).at[rank].set(cbs)
    nd = jnp.sum(first.astype(jnp.int32))
    steps = jnp.arange(T, dtype=jnp.int32)
    flags = (steps < nd).astype(jnp.int32)
    blist = jnp.where(flags == 1, blist, cbs[-1])
    out = _build_matvec(B, V, s.shape[0], T)(blist, flags, x, s, rhs_arr)
    return out.astype(bool)


# speedup vs baseline: 1.4377x; 1.2487x over previous
import functools

import jax
import jax.numpy as jnp
from jax import lax
from jax.experimental import pallas as pl
from jax.experimental.pallas import tpu as pltpu
from jax.experimental.pallas import tpu_sc as plsc

_LANES = 16
_KB = 2048
_NN = 8


@functools.lru_cache(maxsize=None)
def _build_scatter(V, T):
    info = plsc.get_sparse_core_info()
    NC, NS = info.num_cores, info.num_subcores
    NW = NC * NS
    TPW = T // NW
    Vp = ((V + NS * 128 - 1) // (NS * 128)) * NS * 128
    CS = Vp // NS
    assert CS % 128 == 0

    mesh = plsc.VectorSubcoreMesh(core_axis_name="c", subcore_axis_name="s")

    @functools.partial(
        pl.kernel,
        out_type=jax.ShapeDtypeStruct((NC, Vp), jnp.float32),
        mesh=mesh,
        compiler_params=pltpu.CompilerParams(needs_layout_passes=False),
        scratch_types=[
            pltpu.VMEM((TPW,), jnp.int32),
            pltpu.VMEM((TPW,), jnp.float32),
            pltpu.VMEM((CS,), jnp.float32),
            pltpu.VMEM_SHARED((Vp,), jnp.float32),
        ],
    )
    def scatter_kernel(idx_hbm, coeff_hbm, out_hbm, idx_w, coeff_w, zer_v, s_sh):
        cid = lax.axis_index("c")
        sid = lax.axis_index("s")
        row = sid * NC + cid
        pltpu.sync_copy(idx_hbm.at[row], idx_w)
        pltpu.sync_copy(coeff_hbm.at[row], coeff_w)

        z = jnp.zeros((_LANES,), jnp.float32)

        def zero(i, carry):
            zer_v[pl.ds(i * _LANES, _LANES)] = z
            return carry

        lax.fori_loop(0, CS // _LANES, zero, None)
        pltpu.sync_copy(zer_v, s_sh.at[pl.ds(sid * CS, CS)])
        plsc.subcore_barrier()
        pltpu.sync_copy(coeff_w, s_sh.at[idx_w], add=True)
        plsc.subcore_barrier()
        pltpu.sync_copy(s_sh.at[pl.ds(sid * CS, CS)],
                        out_hbm.at[cid].at[pl.ds(sid * CS, CS)])

    return scatter_kernel


@functools.lru_cache(maxsize=None)
def _build_matvec(B, V, NC, Vp):
    grid = Vp // _KB
    tail_valid = V - (grid - 1) * _KB
    NG = _KB // 128

    def accum(xb, sb, acc_ref):
        t = xb[:, 0:128] * sb[:, 0:128]
        for g in range(1, NG):
            t = t + xb[:, g * 128:(g + 1) * 128] * sb[:, g * 128:(g + 1) * 128]
        acc_ref[...] += t

    def body(x_ref, s_ref, rhs_ref, out_ref, acc_ref):
        k = pl.program_id(0)
        sb = s_ref[0:1, :] + s_ref[1:2, :]

        @pl.when(k == 0)
        def _():
            acc_ref[...] = jnp.zeros_like(acc_ref)

        @pl.when(k < grid - 1)
        def _():
            accum(x_ref[...], sb, acc_ref)

        @pl.when(k == grid - 1)
        def _():
            lane = lax.broadcasted_iota(jnp.int32, (B, _KB), 1)
            xb = jnp.where(lane < tail_valid, x_ref[...], 0.0)
            accum(xb, sb, acc_ref)
            lhs = jnp.sum(acc_ref[...], axis=1)
            out_ref[...] = (lhs <= rhs_ref[0]).astype(jnp.int32)

    return pl.pallas_call(
        body,
        grid=(grid,),
        out_shape=jax.ShapeDtypeStruct((B,), jnp.int32),
        in_specs=[
            pl.BlockSpec((B, _KB), lambda k: (0, k)),
            pl.BlockSpec((NC, _KB), lambda k: (0, k)),
            pl.BlockSpec(memory_space=pltpu.SMEM),
        ],
        out_specs=pl.BlockSpec((B,), lambda k: (0,)),
        scratch_shapes=[pltpu.VMEM((B, 128), jnp.float32)],
        compiler_params=pltpu.CompilerParams(
            dimension_semantics=("arbitrary",)),
    )


def kernel(x, coeff_tensor, indices_tensor, rhs):
    B, V = x.shape
    T = indices_tensor.shape[0]
    info = plsc.get_sparse_core_info()
    NW = info.num_cores * info.num_subcores
    idx2 = indices_tensor.reshape(NW, T // NW)
    coeff2 = coeff_tensor.reshape(NW, T // NW)
    s = _build_scatter(V, T)(idx2, coeff2)
    rhs_arr = jnp.full((1,), rhs, dtype=jnp.float32)
    out = _build_matvec(B, V, s.shape[0], s.shape[1])(x, s, rhs_arr)
    return out.astype(bool)
